# Initial kernel scaffold; baseline (speedup 1.0000x reference)
import jax
import jax.numpy as jnp
from jax import lax
from jax.experimental import pallas as pl
from jax.experimental.pallas import tpu as pltpu
from jax.experimental.pallas import tpu_sc as plsc

N = 10000
E = 320000
EPS = 1e-5

NC = 2
NS = 16
NW = NC * NS
CHUNK = 128
CPW = -(-E // (NW * CHUNK))
E_PAD = NW * CHUNK * CPW
ACC_ROWS = 10240
TRASH = N
RPS = ACC_ROWS // NS
OPS = N // NS

_mesh = plsc.VectorSubcoreMesh(
    core_axis_name="c", subcore_axis_name="s", num_cores=NC, num_subcores=NS)


def _deg_body(dst_hbm, cnt_out, zbuf, idx_all, acc):
    c = lax.axis_index("c")
    s = lax.axis_index("s")
    wid = s * NC + c
    z16 = jnp.zeros((16,), jnp.float32)

    def zrow(i, _):
        zbuf[i, :] = z16
        return 0
    lax.fori_loop(0, CHUNK, zrow, 0)
    for b in range(RPS // CHUNK):
        pltpu.sync_copy(zbuf, acc.at[pl.ds(s * RPS + b * CHUNK, CHUNK)])
    plsc.subcore_barrier()

    one16 = jnp.full((16,), 1.0, jnp.float32)

    def orow(i, _):
        zbuf[i, :] = one16
        return 0
    lax.fori_loop(0, CHUNK, orow, 0)

    pltpu.sync_copy(dst_hbm.at[wid], idx_all)

    def chunk(k, _):
        pltpu.sync_copy(zbuf, acc.at[idx_all.at[k]], add=True)
        return 0
    lax.fori_loop(0, CPW, chunk, 0)
    plsc.subcore_barrier()
    pltpu.sync_copy(acc.at[pl.ds(s * OPS, OPS)],
                    cnt_out.at[c, pl.ds(s * OPS, OPS)])


_deg = pl.kernel(
    _deg_body,
    out_type=jax.ShapeDtypeStruct((NC, N, 16), jnp.float32),
    mesh=_mesh,
    scratch_types=[
        pltpu.VMEM((CHUNK, 16), jnp.float32),
        pltpu.VMEM((CPW, CHUNK), jnp.int32),
        pltpu.VMEM_SHARED((ACC_ROWS, 16), jnp.float32),
    ],
)


def _agg_body(u_hbm, src_hbm, dst_hbm, part_out, rows_v, isrc, idst, sem, acc):
    c = lax.axis_index("c")
    s = lax.axis_index("s")
    wid = s * NC + c
    z16 = jnp.zeros((16,), jnp.float32)

    def zrow(i, _):
        for j in range(8):
            rows_v[i, pl.ds(j * 16, 16)] = z16
        return 0
    lax.fori_loop(0, CHUNK, zrow, 0)
    for b in range(RPS // CHUNK):
        pltpu.sync_copy(rows_v, acc.at[pl.ds(s * RPS + b * CHUNK, CHUNK)])
    plsc.subcore_barrier()

    pltpu.sync_copy(src_hbm.at[wid], isrc)
    pltpu.sync_copy(dst_hbm.at[wid], idst)

    def chunk(k, _):
        pltpu.async_copy(u_hbm.at[isrc.at[k]], rows_v, sem).wait()
        pltpu.sync_copy(rows_v, acc.at[idst.at[k]], add=True)
        return 0
    lax.fori_loop(0, CPW, chunk, 0)
    plsc.subcore_barrier()
    pltpu.sync_copy(acc.at[pl.ds(s * OPS, OPS)],
                    part_out.at[c, pl.ds(s * OPS, OPS)])


_agg = pl.kernel(
    _agg_body,
    out_type=jax.ShapeDtypeStruct((NC, N, 128), jnp.float32),
    mesh=_mesh,
    scratch_types=[
        pltpu.VMEM((CHUNK, 128), jnp.float32),
        pltpu.VMEM((CPW, CHUNK), jnp.int32),
        pltpu.VMEM((CPW, CHUNK), jnp.int32),
        pltpu.SemaphoreType.DMA,
        pltpu.VMEM_SHARED((ACC_ROWS, 128), jnp.float32),
    ],
)


def _tcA_body(x_ref, w0_ref, b0_ref, w1_ref, c0_ref, c1_ref, u1_ref, dinv_ref):
    t = jnp.dot(x_ref[:], w0_ref[:], preferred_element_type=jnp.float32)
    t = t + b0_ref[:]
    t1 = jnp.dot(t, w1_ref[:], preferred_element_type=jnp.float32)
    deg = c0_ref[:] + c1_ref[:] + 1.0
    dinv = lax.rsqrt(deg)
    dinv_ref[:] = dinv
    u1_ref[:] = dinv * t1


def _bn_relu(y, g, be):
    m = jnp.mean(y, axis=0, keepdims=True)
    d = y - m
    v = jnp.mean(d * d, axis=0, keepdims=True)
    h = g * d * lax.rsqrt(v + EPS) + be
    return jnp.maximum(h, 0.0)


def _tcB_body(p0_ref, p1_ref, u1_ref, dinv_ref, b1_ref, g1_ref, be1_ref,
              w2_ref, u2_ref):
    dinv = dinv_ref[:]
    y = dinv * (p0_ref[:] + p1_ref[:] + u1_ref[:]) + b1_ref[:]
    h = _bn_relu(y, g1_ref[:], be1_ref[:])
    t2 = jnp.dot(h, w2_ref[:], preferred_element_type=jnp.float32)
    u2_ref[:] = dinv * t2


def _tcC_body(p0_ref, p1_ref, u2_ref, dinv_ref, b2_ref, g2_ref, be2_ref,
              wout_ref, bout_ref, out_ref):
    y = dinv_ref[:] * (p0_ref[:] + p1_ref[:] + u2_ref[:]) + b2_ref[:]
    h = _bn_relu(y, g2_ref[:], be2_ref[:])
    logits = jnp.dot(h, wout_ref[:], preferred_element_type=jnp.float32)
    logits = logits + bout_ref[:]
    mx = jnp.max(logits, axis=1, keepdims=True)
    lse = jnp.log(jnp.sum(jnp.exp(logits - mx), axis=1, keepdims=True)) + mx
    out_ref[:] = logits - lse


_tcA = pl.pallas_call(
    _tcA_body,
    out_shape=[jax.ShapeDtypeStruct((N, 128), jnp.float32),
               jax.ShapeDtypeStruct((N, 1), jnp.float32)])

_tcB = pl.pallas_call(
    _tcB_body,
    out_shape=jax.ShapeDtypeStruct((N, 128), jnp.float32))

_tcC = pl.pallas_call(
    _tcC_body,
    out_shape=jax.ShapeDtypeStruct((N, 16), jnp.float32))


def kernel(x, edge_index, W0, b0, W1, b1, W2, b2, g1, be1, g2, be2, Wout, bout):
    src = edge_index[0].astype(jnp.int32)
    dst = edge_index[1].astype(jnp.int32)
    pad = E_PAD - E
    srcp = jnp.concatenate([src, jnp.zeros((pad,), jnp.int32)])
    dstp = jnp.concatenate([dst, jnp.full((pad,), TRASH, jnp.int32)])
    src3 = srcp.reshape(NW, CPW, CHUNK)
    dst3 = dstp.reshape(NW, CPW, CHUNK)

    cnt = _deg(dst3)
    c0 = cnt[0, :, 0:1]
    c1 = cnt[1, :, 0:1]
    u1, dinv = _tcA(x, W0, b0.reshape(1, -1), W1, c0, c1)
    part1 = _agg(u1, src3, dst3)
    u2 = _tcB(part1[0], part1[1], u1, dinv, b1.reshape(1, -1),
              g1.reshape(1, -1), be1.reshape(1, -1), W2)
    part2 = _agg(u2, src3, dst3)
    out = _tcC(part2[0], part2[1], u2, dinv, b2.reshape(1, -1),
               g2.reshape(1, -1), be2.reshape(1, -1), Wout,
               bout.reshape(1, -1))
    return out


# speedup vs baseline: 12.7126x; 12.7126x over previous
import functools

import jax
import jax.numpy as jnp
from jax import lax
from jax.experimental import pallas as pl
from jax.experimental.pallas import tpu as pltpu
from jax.experimental.pallas import tpu_sc as plsc

N = 10000
E = 320000
EPS = 1e-5

NC = 2
NS = 16
NW = NC * NS
CHUNK = 128
CPW = -(-E // (NW * CHUNK))
E_PAD = NW * CHUNK * CPW
ACC_ROWS = 10240
TRASH = N
RPS = ACC_ROWS // NS

def _deg_body(dst_hbm, cnt_out, zbuf, idx_all, acc):
    c = lax.axis_index("c")
    s = lax.axis_index("s")
    wid = s * NC + c
    z16 = jnp.zeros((16,), jnp.float32)

    def zrow(i, _):
        zbuf[i, :] = z16
        return 0
    lax.fori_loop(0, CHUNK, zrow, 0)
    for b in range(RPS // CHUNK):
        pltpu.sync_copy(zbuf, acc.at[pl.ds(s * RPS + b * CHUNK, CHUNK)])
    plsc.subcore_barrier()

    one16 = jnp.full((16,), 1.0, jnp.float32)

    def orow(i, _):
        zbuf[i, :] = one16
        return 0
    lax.fori_loop(0, CHUNK, orow, 0)

    pltpu.sync_copy(dst_hbm.at[wid], idx_all)

    def chunk(k, _):
        pltpu.sync_copy(zbuf, acc.at[idx_all.at[k]], add=True)
        return 0
    lax.fori_loop(0, CPW, chunk, 0)
    plsc.subcore_barrier()
    pltpu.sync_copy(acc.at[pl.ds(s * RPS, RPS)],
                    cnt_out.at[c, pl.ds(s * RPS, RPS)])


@functools.cache
def _get_deg():
    mesh = plsc.VectorSubcoreMesh(
        core_axis_name="c", subcore_axis_name="s",
        num_cores=NC, num_subcores=NS)
    return pl.kernel(
        _deg_body,
        out_type=jax.ShapeDtypeStruct((NC, ACC_ROWS, 16), jnp.float32),
        mesh=mesh,
        scratch_types=[
            pltpu.VMEM((CHUNK, 16), jnp.float32),
            pltpu.VMEM((CPW, CHUNK), jnp.int32),
            pltpu.VMEM_SHARED((ACC_ROWS, 16), jnp.float32),
        ],
    )


def _agg_body(u_hbm, src_hbm, dst_hbm, part_out, rows_v, isrc, idst, sem, acc):
    c = lax.axis_index("c")
    s = lax.axis_index("s")
    wid = s * NC + c
    z16 = jnp.zeros((16,), jnp.float32)

    def zrow(i, _):
        for j in range(8):
            rows_v[i, pl.ds(j * 16, 16)] = z16
        return 0
    lax.fori_loop(0, CHUNK, zrow, 0)
    for b in range(RPS // CHUNK):
        pltpu.sync_copy(rows_v, acc.at[pl.ds(s * RPS + b * CHUNK, CHUNK)])
    plsc.subcore_barrier()

    pltpu.sync_copy(src_hbm.at[wid], isrc)
    pltpu.sync_copy(dst_hbm.at[wid], idst)

    def chunk(k, _):
        pltpu.async_copy(u_hbm.at[isrc.at[k]], rows_v, sem).wait()
        pltpu.sync_copy(rows_v, acc.at[idst.at[k]], add=True)
        return 0
    lax.fori_loop(0, CPW, chunk, 0)
    plsc.subcore_barrier()
    pltpu.sync_copy(acc.at[pl.ds(s * RPS, RPS)],
                    part_out.at[c, pl.ds(s * RPS, RPS)])


@functools.cache
def _get_agg():
    mesh = plsc.VectorSubcoreMesh(
        core_axis_name="c", subcore_axis_name="s",
        num_cores=NC, num_subcores=NS)
    return pl.kernel(
        _agg_body,
        out_type=jax.ShapeDtypeStruct((NC, ACC_ROWS, 128), jnp.float32),
        mesh=mesh,
        scratch_types=[
            pltpu.VMEM((CHUNK, 128), jnp.float32),
            pltpu.VMEM((CPW, CHUNK), jnp.int32),
            pltpu.VMEM((CPW, CHUNK), jnp.int32),
            pltpu.SemaphoreType.DMA,
            pltpu.VMEM_SHARED((ACC_ROWS, 128), jnp.float32),
        ],
    )


def _tcA_body(x_ref, w0_ref, b0_ref, w1_ref, c0_ref, c1_ref, u1_ref, dinv_ref):
    t = jnp.dot(x_ref[:], w0_ref[:], preferred_element_type=jnp.float32)
    t = t + b0_ref[:]
    t1 = jnp.dot(t, w1_ref[:], preferred_element_type=jnp.float32)
    deg = c0_ref[:] + c1_ref[:] + 1.0
    dinv = lax.rsqrt(deg)
    dinv_ref[:] = dinv
    u1_ref[:] = dinv * t1


def _bn_relu(y, g, be):
    m = jnp.mean(y, axis=0, keepdims=True)
    d = y - m
    v = jnp.mean(d * d, axis=0, keepdims=True)
    h = g * d * lax.rsqrt(v + EPS) + be
    return jnp.maximum(h, 0.0)


def _tcB_body(p0_ref, p1_ref, u1_ref, dinv_ref, b1_ref, g1_ref, be1_ref,
              w2_ref, u2_ref):
    dinv = dinv_ref[:]
    y = dinv * (p0_ref[:] + p1_ref[:] + u1_ref[:]) + b1_ref[:]
    h = _bn_relu(y, g1_ref[:], be1_ref[:])
    t2 = jnp.dot(h, w2_ref[:], preferred_element_type=jnp.float32)
    u2_ref[:] = dinv * t2


def _tcC_body(p0_ref, p1_ref, u2_ref, dinv_ref, b2_ref, g2_ref, be2_ref,
              wout_ref, bout_ref, out_ref):
    y = dinv_ref[:] * (p0_ref[:] + p1_ref[:] + u2_ref[:]) + b2_ref[:]
    h = _bn_relu(y, g2_ref[:], be2_ref[:])
    logits = jnp.dot(h, wout_ref[:], preferred_element_type=jnp.float32)
    logits = logits + bout_ref[:]
    mx = jnp.max(logits, axis=1, keepdims=True)
    lse = jnp.log(jnp.sum(jnp.exp(logits - mx), axis=1, keepdims=True)) + mx
    out_ref[:] = logits - lse


_tcA = pl.pallas_call(
    _tcA_body,
    out_shape=[jax.ShapeDtypeStruct((N, 128), jnp.float32),
               jax.ShapeDtypeStruct((N, 1), jnp.float32)])

_tcB = pl.pallas_call(
    _tcB_body,
    out_shape=jax.ShapeDtypeStruct((N, 128), jnp.float32))

_tcC = pl.pallas_call(
    _tcC_body,
    out_shape=jax.ShapeDtypeStruct((N, 16), jnp.float32))


def kernel(x, edge_index, W0, b0, W1, b1, W2, b2, g1, be1, g2, be2, Wout, bout):
    src = edge_index[0].astype(jnp.int32)
    dst = edge_index[1].astype(jnp.int32)
    pad = E_PAD - E
    srcp = jnp.concatenate([src, jnp.zeros((pad,), jnp.int32)])
    dstp = jnp.concatenate([dst, jnp.full((pad,), TRASH, jnp.int32)])
    src3 = srcp.reshape(NW, CPW, CHUNK)
    dst3 = dstp.reshape(NW, CPW, CHUNK)

    cnt = _get_deg()(dst3)
    c0 = cnt[0, :N, 0:1]
    c1 = cnt[1, :N, 0:1]
    u1, dinv = _tcA(x, W0, b0.reshape(1, -1), W1, c0, c1)
    agg = _get_agg()
    part1 = agg(u1, src3, dst3)
    u2 = _tcB(part1[0, :N], part1[1, :N], u1, dinv, b1.reshape(1, -1),
              g1.reshape(1, -1), be1.reshape(1, -1), W2)
    part2 = agg(u2, src3, dst3)
    out = _tcC(part2[0, :N], part2[1, :N], u2, dinv, b2.reshape(1, -1),
               g2.reshape(1, -1), be2.reshape(1, -1), Wout,
               bout.reshape(1, -1))
    return out
